# baseline (speedup 1.0000x reference)
import functools

import jax
import jax.numpy as jnp
from jax import lax
from jax.experimental import pallas as pl
from jax.experimental.pallas import tpu as pltpu
from jax.experimental.pallas import tpu_sc as plsc

_NC = 1
_NS = 16
_NW = _NC * _NS
_L = 16
_NEG = -1.0e30
_PW = 272


def _prep_body(alf, arf, alb, arb, wf, wb, out):
    dn = (((1,), (1,)), ((), ()))
    out[0:1, :] = lax.dot_general(alf[:], wf[:], dn, preferred_element_type=jnp.float32)
    out[1:2, :] = lax.dot_general(arf[:], wf[:], dn, preferred_element_type=jnp.float32)
    out[2:3, :] = lax.dot_general(alb[:], wb[:], dn, preferred_element_type=jnp.float32)
    out[3:4, :] = lax.dot_general(arb[:], wb[:], dn, preferred_element_type=jnp.float32)


def _make_scan(E, N, D):
    EPW = E // _NW
    BLK = 25
    NB = EPW // (BLK * _L)
    assert EPW * _NW == E and NB * BLK * _L == EPW and D == 128
    NCH = D // _L

    def body(src_hbm, dst_hbm, feat_hbm, wv_hbm, op_hbm, par_hbm,
             wf_hbm, wb_hbm, bf_hbm, bb_hbm,
             final, parts,
             srcb, dstb, flist_f, flist_b, wvbuf, opbuf, idxs, idxd,
             rows_s, rows_d, accbuf, mdbuf, parbuf, parrows,
             wmat, bbuf, pbuf, wbuf, outbuf, scal, cnts, sem, sem2):
        tid = lax.axis_index("s")
        base = tid * EPW

        stage1 = pltpu.async_copy(src_hbm.at[pl.ds(base, EPW)], srcb, sem2)
        stage2 = pltpu.async_copy(dst_hbm.at[pl.ds(base, EPW)], dstb, sem2)
        pltpu.sync_copy(wv_hbm, wvbuf)
        pltpu.sync_copy(op_hbm, opbuf)

        opv = opbuf[...]
        opn = opv[0]
        idxs[:] = opv
        pltpu.async_copy(feat_hbm.at[idxs], rows_s, sem).wait()

        def dot_row(ref_a, ia, wbase):
            a = ref_a[ia, pl.ds(0, _L)] * wvbuf[pl.ds(wbase, _L)]
            for c in range(1, NCH):
                a = a + ref_a[ia, pl.ds(c * _L, _L)] * wvbuf[pl.ds(wbase + c * _L, _L)]
            return jnp.sum(a)

        er_f = dot_row(rows_s, 0, D)
        er_b = dot_row(rows_s, 0, 3 * D)

        @pl.when(tid == 2)
        def _():
            for c in range(NCH):
                outbuf[pl.ds(c * _L, _L)] = rows_s[0, pl.ds(c * _L, _L)]

        scal[0] = _NEG
        scal[1] = 0.0
        scal[2] = _NEG
        scal[3] = 0.0
        cnts[0] = 0
        cnts[1] = 0
        for c in range(2 * NCH):
            accbuf[pl.ds(c * _L, _L)] = jnp.zeros((_L,), jnp.float32)

        def exp_scalar(x):
            return jnp.max(jnp.exp(jnp.full((_L,), x, jnp.float32)))

        def online_update(dirn, s, rowref, j):
            m_old = scal[2 * dirn]
            d_old = scal[2 * dirn + 1]

            @pl.when(s <= m_old)
            def _():
                w = exp_scalar(s - m_old)
                scal[2 * dirn + 1] = d_old + w
                for c in range(NCH):
                    ds_ = pl.ds(dirn * D + c * _L, _L)
                    accbuf[ds_] = accbuf[ds_] + w * rowref[j, pl.ds(c * _L, _L)]

            @pl.when(s > m_old)
            def _():
                sc = exp_scalar(m_old - s)
                scal[2 * dirn] = s
                scal[2 * dirn + 1] = d_old * sc + 1.0
                for c in range(NCH):
                    ds_ = pl.ds(dirn * D + c * _L, _L)
                    accbuf[ds_] = accbuf[ds_] * sc + rowref[j, pl.ds(c * _L, _L)]

        stage1.wait()
        stage2.wait()

        def record_vreg(off):
            sv = srcb[pl.ds(off, _L)]
            dv = dstb[pl.ds(off, _L)]
            mf = dv == opv
            mb = sv == opv
            cf = cnts[0]
            cb = cnts[1]
            plsc.store_compressed(flist_f.at[pl.ds(cf, _L)], sv, mask=mf)
            plsc.store_compressed(flist_b.at[pl.ds(cb, _L)], dv, mask=mb)
            cnts[0] = cf + plsc.all_reduce_population_count(mf)[0]
            cnts[1] = cb + plsc.all_reduce_population_count(mb)[0]

        def block(b, carry):
            bbase = b * (BLK * _L)
            hit = srcb[pl.ds(bbase, _L)] == opv
            hit = hit | (dstb[pl.ds(bbase, _L)] == opv)
            for u in range(1, BLK):
                hit = hit | (srcb[pl.ds(bbase + u * _L, _L)] == opv)
                hit = hit | (dstb[pl.ds(bbase + u * _L, _L)] == opv)
            cnt = jnp.sum(jnp.where(hit, 1, 0))

            @pl.when(cnt > 0)
            def _():
                def rescan(u, c2):
                    record_vreg(bbase + u * _L)
                    return c2
                lax.fori_loop(0, BLK, rescan, 0)
            return carry

        lax.fori_loop(0, NB, block, 0)

        lane = lax.broadcasted_iota(jnp.int32, (_L,), 0)

        def post_pass(dirn, flist, idxbuf, rows, wbase, er):
            cnt = cnts[dirn]

            def chunk(ci, carry):
                i = ci * _L
                raw = flist[pl.ds(i, _L)]
                valid = (lane + i) < jnp.full((_L,), cnt, jnp.int32)
                idxbuf[:] = jnp.where(valid, raw, 0)
                pltpu.async_copy(feat_hbm.at[idxbuf], rows, sem).wait()
                for j in range(_L):
                    @pl.when(i + j < cnt)
                    def _():
                        x = dot_row(rows, j, wbase) + er
                        online_update(dirn, jnp.where(x >= 0.0, x, 0.2 * x),
                                      rows, j)
                return carry

            lax.fori_loop(0, (cnt + _L - 1) // _L, chunk, 0)

        post_pass(0, flist_f, idxs, rows_s, 0, er_f)
        post_pass(1, flist_b, idxd, rows_d, 2 * D, er_b)

        mdv = jnp.zeros((_L,), jnp.float32)
        for k in range(4):
            mdv = jnp.where(lane == k, scal[k], mdv)
        mdbuf[:] = mdv
        pltpu.sync_copy(mdbuf, parts.at[pl.ds(tid * _PW, _L)])
        pltpu.sync_copy(accbuf, parts.at[pl.ds(tid * _PW + _L, 2 * D)])
        plsc.subcore_barrier()

        @pl.when(tid < 2)
        def _():
            @pl.when(tid == 0)
            def _():
                pltpu.sync_copy(wf_hbm, wmat)
                pltpu.sync_copy(bf_hbm, bbuf)

            @pl.when(tid == 1)
            def _():
                pltpu.sync_copy(wb_hbm, wmat)
                pltpu.sync_copy(bb_hbm, bbuf)

            pltpu.sync_copy(parts, pbuf)
            accoff = _L + tid * D

            ms, ds_l = [], []
            for t in range(_NW):
                mv = pbuf[pl.ds(t * _PW, _L)]
                ms.append(jnp.where(tid == 0, mv[0], mv[2]))
                ds_l.append(jnp.where(tid == 0, mv[1], mv[3]))
            mx = ms[0]
            for t in range(1, _NW):
                mx = jnp.maximum(mx, ms[t])
            ws = [exp_scalar(ms[t] - mx) for t in range(_NW)]
            dtot = ds_l[0] * ws[0]
            for t in range(1, _NW):
                dtot = dtot + ds_l[t] * ws[t]
            dvec = jnp.full((_L,), dtot, jnp.float32)
            inv = jnp.where(dvec > 0.0, jnp.ones((_L,), jnp.float32) / dvec,
                            jnp.zeros((_L,), jnp.float32))
            for c in range(NCH):
                a = pbuf[pl.ds(accoff + c * _L, _L)] * ws[0]
                for t in range(1, _NW):
                    a = a + pbuf[pl.ds(t * _PW + accoff + c * _L, _L)] * ws[t]
                wbuf[pl.ds(c * _L, _L)] = a * inv

            def mv_step(k, acc):
                wk = wbuf[pl.ds(k, _L)][0]
                return tuple(acc[c] + wk * wmat[pl.ds(k * D + c * _L, _L)]
                             for c in range(NCH))

            zero = jnp.zeros((_L,), jnp.float32)
            fh = lax.fori_loop(0, D, mv_step, (zero,) * NCH)
            for c in range(NCH):
                outbuf[pl.ds(c * _L, _L)] = fh[c] + bbuf[pl.ds(c * _L, _L)]
            pltpu.sync_copy(outbuf, final.at[pl.ds(tid * D, D)])

        @pl.when(tid == 2)
        def _():
            pltpu.sync_copy(outbuf, final.at[pl.ds(2 * D, D)])

        @pl.when(tid == 3)
        def _():
            pltpu.sync_copy(par_hbm, parbuf)
            pltpu.async_copy(feat_hbm.at[parbuf], parrows, sem).wait()
            for c in range(NCH):
                s = parrows[0, pl.ds(c * _L, _L)]
                for r in range(1, parrows.shape[0]):
                    s = s + parrows[r, pl.ds(c * _L, _L)]
                outbuf[pl.ds(c * _L, _L)] = s
            pltpu.sync_copy(outbuf, final.at[pl.ds(3 * D, D)])

    mesh = plsc.VectorSubcoreMesh(core_axis_name="c", subcore_axis_name="s",
                                  num_cores=_NC, num_subcores=_NS)
    return pl.kernel(
        body,
        out_type=(
            jax.ShapeDtypeStruct((4 * D,), jnp.float32),
            jax.ShapeDtypeStruct((_NW * _PW,), jnp.float32),
        ),
        mesh=mesh,
        compiler_params=pltpu.CompilerParams(needs_layout_passes=False),
        scratch_types=[
            pltpu.VMEM((EPW,), jnp.int32),
            pltpu.VMEM((EPW,), jnp.int32),
            pltpu.VMEM((EPW + _L,), jnp.int32),
            pltpu.VMEM((EPW + _L,), jnp.int32),
            pltpu.VMEM((4 * D,), jnp.float32),
            pltpu.VMEM((_L,), jnp.int32),
            pltpu.VMEM((_L,), jnp.int32),
            pltpu.VMEM((_L,), jnp.int32),
            pltpu.VMEM((_L, D), jnp.float32),
            pltpu.VMEM((_L, D), jnp.float32),
            pltpu.VMEM((2 * D,), jnp.float32),
            pltpu.VMEM((_L,), jnp.float32),
            pltpu.VMEM((8,), jnp.int32),
            pltpu.VMEM((8, D), jnp.float32),
            pltpu.VMEM((D * D,), jnp.float32),
            pltpu.VMEM((D,), jnp.float32),
            pltpu.VMEM((_NW * _PW,), jnp.float32),
            pltpu.VMEM((D + _L,), jnp.float32),
            pltpu.VMEM((D,), jnp.float32),
            pltpu.SMEM((8,), jnp.float32),
            pltpu.SMEM((8,), jnp.int32),
            pltpu.SemaphoreType.DMA,
            pltpu.SemaphoreType.DMA,
        ],
    )


def kernel(feat, edge_index, op, parallel, W_f, al_f, ar_f, b_f,
           W_b, al_b, ar_b, b_b):
    N, D = feat.shape
    E = edge_index.shape[1]
    H = al_f.shape[0]
    assert H == 1 and D == 128

    wv = pl.pallas_call(
        _prep_body,
        out_shape=jax.ShapeDtypeStruct((4, D), jnp.float32),
    )(al_f, ar_f, al_b, ar_b, W_f, W_b)

    src = edge_index[0]
    dst = edge_index[1]
    op_arr = jnp.full((_L,), op, dtype=jnp.int32)
    par = parallel.astype(jnp.int32)

    final, _ = _make_scan(E, N, D)(
        src, dst, feat, wv.reshape(4 * D), op_arr, par,
        W_f.reshape(D * D), W_b.reshape(D * D), b_f, b_b)
    return final


# speedup vs baseline: 1.0233x; 1.0233x over previous
import functools

import jax
import jax.numpy as jnp
from jax import lax
from jax.experimental import pallas as pl
from jax.experimental.pallas import tpu as pltpu
from jax.experimental.pallas import tpu_sc as plsc

_NC = 1
_NS = 16
_NW = _NC * _NS
_L = 16
_NEG = -1.0e30
_PW = 272


def _prep_body(alf, arf, alb, arb, wf, wb, out):
    dn = (((1,), (1,)), ((), ()))
    out[0:1, :] = lax.dot_general(alf[:], wf[:], dn, preferred_element_type=jnp.float32)
    out[1:2, :] = lax.dot_general(arf[:], wf[:], dn, preferred_element_type=jnp.float32)
    out[2:3, :] = lax.dot_general(alb[:], wb[:], dn, preferred_element_type=jnp.float32)
    out[3:4, :] = lax.dot_general(arb[:], wb[:], dn, preferred_element_type=jnp.float32)


def _make_scan(E, N, D):
    EPW = E // _NW
    BLK = 25
    NB = EPW // (BLK * _L)
    assert EPW * _NW == E and NB * BLK * _L == EPW and D == 128
    NCH = D // _L

    def body(src_hbm, dst_hbm, feat_hbm, wv_hbm, op_hbm, par_hbm,
             wf_hbm, wb_hbm, bf_hbm, bb_hbm,
             final, parts,
             srcb, dstb, flist_f, flist_b, wvbuf, opbuf, idxs, idxd,
             rows_s, rows_d, accbuf, mdbuf, parbuf, parrows,
             wmat, bbuf, pbuf, wbuf, outbuf, scal, cnts, sem, sem2):
        tid = lax.axis_index("s")
        base = tid * EPW

        stage1 = pltpu.async_copy(src_hbm.at[pl.ds(base, EPW)], srcb, sem2)
        stage2 = pltpu.async_copy(dst_hbm.at[pl.ds(base, EPW)], dstb, sem2)
        pltpu.sync_copy(wv_hbm, wvbuf)
        pltpu.sync_copy(op_hbm, opbuf)

        opv = opbuf[...]
        opn = opv[0]
        idxs[:] = opv
        pltpu.async_copy(feat_hbm.at[idxs], rows_s, sem).wait()

        def dot_row(ref_a, ia, wbase):
            a = ref_a[ia, pl.ds(0, _L)] * wvbuf[pl.ds(wbase, _L)]
            for c in range(1, NCH):
                a = a + ref_a[ia, pl.ds(c * _L, _L)] * wvbuf[pl.ds(wbase + c * _L, _L)]
            return jnp.sum(a)

        er_f = dot_row(rows_s, 0, D)
        er_b = dot_row(rows_s, 0, 3 * D)

        @pl.when(tid == 2)
        def _():
            for c in range(NCH):
                outbuf[pl.ds(c * _L, _L)] = rows_s[0, pl.ds(c * _L, _L)]

        scal[0] = _NEG
        scal[1] = 0.0
        scal[2] = _NEG
        scal[3] = 0.0
        cnts[0] = 0
        cnts[1] = 0
        for c in range(2 * NCH):
            accbuf[pl.ds(c * _L, _L)] = jnp.zeros((_L,), jnp.float32)

        def exp_scalar(x):
            return jnp.max(jnp.exp(jnp.full((_L,), x, jnp.float32)))

        def online_update(dirn, s, rowref, j):
            m_old = scal[2 * dirn]
            d_old = scal[2 * dirn + 1]

            @pl.when(s <= m_old)
            def _():
                w = exp_scalar(s - m_old)
                scal[2 * dirn + 1] = d_old + w
                for c in range(NCH):
                    ds_ = pl.ds(dirn * D + c * _L, _L)
                    accbuf[ds_] = accbuf[ds_] + w * rowref[j, pl.ds(c * _L, _L)]

            @pl.when(s > m_old)
            def _():
                sc = exp_scalar(m_old - s)
                scal[2 * dirn] = s
                scal[2 * dirn + 1] = d_old * sc + 1.0
                for c in range(NCH):
                    ds_ = pl.ds(dirn * D + c * _L, _L)
                    accbuf[ds_] = accbuf[ds_] * sc + rowref[j, pl.ds(c * _L, _L)]

        stage1.wait()
        stage2.wait()

        def record_vreg(off):
            sv = srcb[pl.ds(off, _L)]
            dv = dstb[pl.ds(off, _L)]
            mf = dv == opv
            mb = sv == opv
            cf = cnts[0]
            cb = cnts[1]
            plsc.store_compressed(flist_f.at[pl.ds(cf, _L)], sv, mask=mf)
            plsc.store_compressed(flist_b.at[pl.ds(cb, _L)], dv, mask=mb)
            cnts[0] = cf + plsc.all_reduce_population_count(mf)[0]
            cnts[1] = cb + plsc.all_reduce_population_count(mb)[0]

        def block(b, carry):
            bbase = b * (BLK * _L)
            hit = srcb[pl.ds(bbase, _L)] == opv
            hit = hit | (dstb[pl.ds(bbase, _L)] == opv)
            for u in range(1, BLK):
                hit = hit | (srcb[pl.ds(bbase + u * _L, _L)] == opv)
                hit = hit | (dstb[pl.ds(bbase + u * _L, _L)] == opv)
            cnt = jnp.sum(jnp.where(hit, 1, 0))

            @pl.when(cnt > 0)
            def _():
                def rescan(u, c2):
                    record_vreg(bbase + u * _L)
                    return c2
                lax.fori_loop(0, BLK, rescan, 0)
            return carry

        lax.fori_loop(0, NB, block, 0)

        lane = lax.broadcasted_iota(jnp.int32, (_L,), 0)

        def post_pass(dirn, flist, idxbuf, rows, wbase, er):
            cnt = cnts[dirn]

            def chunk(ci, carry):
                i = ci * _L
                raw = flist[pl.ds(i, _L)]
                valid = (lane + i) < jnp.full((_L,), cnt, jnp.int32)
                idxbuf[:] = jnp.where(valid, raw, 0)
                pltpu.async_copy(feat_hbm.at[idxbuf], rows, sem).wait()
                for j in range(_L):
                    @pl.when(i + j < cnt)
                    def _():
                        x = dot_row(rows, j, wbase) + er
                        online_update(dirn, jnp.where(x >= 0.0, x, 0.2 * x),
                                      rows, j)
                return carry

            lax.fori_loop(0, (cnt + _L - 1) // _L, chunk, 0)

        post_pass(0, flist_f, idxs, rows_s, 0, er_f)
        post_pass(1, flist_b, idxd, rows_d, 2 * D, er_b)

        mdv = jnp.zeros((_L,), jnp.float32)
        for k in range(4):
            mdv = jnp.where(lane == k, scal[k], mdv)
        mdbuf[:] = mdv
        pltpu.sync_copy(mdbuf, parts.at[pl.ds(tid * _PW, _L)])
        pltpu.sync_copy(accbuf, parts.at[pl.ds(tid * _PW + _L, 2 * D)])
        plsc.subcore_barrier()

        @pl.when(tid < 2)
        def _():
            @pl.when(tid == 0)
            def _():
                pltpu.sync_copy(wf_hbm, wmat)
                pltpu.sync_copy(bf_hbm, bbuf)

            @pl.when(tid == 1)
            def _():
                pltpu.sync_copy(wb_hbm, wmat)
                pltpu.sync_copy(bb_hbm, bbuf)

            pltpu.sync_copy(parts, pbuf)
            accoff = _L + tid * D

            ms, ds_l = [], []
            for t in range(_NW):
                mv = pbuf[pl.ds(t * _PW, _L)]
                ms.append(jnp.where(tid == 0, mv[0], mv[2]))
                ds_l.append(jnp.where(tid == 0, mv[1], mv[3]))
            mx = ms[0]
            for t in range(1, _NW):
                mx = jnp.maximum(mx, ms[t])
            ws = [exp_scalar(ms[t] - mx) for t in range(_NW)]
            dtot = ds_l[0] * ws[0]
            for t in range(1, _NW):
                dtot = dtot + ds_l[t] * ws[t]
            dvec = jnp.full((_L,), dtot, jnp.float32)
            inv = jnp.where(dvec > 0.0, jnp.ones((_L,), jnp.float32) / dvec,
                            jnp.zeros((_L,), jnp.float32))
            for c in range(NCH):
                a = pbuf[pl.ds(accoff + c * _L, _L)] * ws[0]
                for t in range(1, _NW):
                    a = a + pbuf[pl.ds(t * _PW + accoff + c * _L, _L)] * ws[t]
                wbuf[pl.ds(c * _L, _L)] = a * inv

            def mv_step(k, acc):
                wk = wbuf[pl.ds(k, _L)][0]
                return tuple(acc[c] + wk * wmat[pl.ds(k * D + c * _L, _L)]
                             for c in range(NCH))

            zero = jnp.zeros((_L,), jnp.float32)
            fh = lax.fori_loop(0, D, mv_step, (zero,) * NCH)
            for c in range(NCH):
                outbuf[pl.ds(c * _L, _L)] = fh[c] + bbuf[pl.ds(c * _L, _L)]
            pltpu.sync_copy(outbuf, final.at[pl.ds(tid * D, D)])

        @pl.when(tid == 2)
        def _():
            pltpu.sync_copy(outbuf, final.at[pl.ds(2 * D, D)])

        @pl.when(tid == 3)
        def _():
            pltpu.sync_copy(par_hbm, parbuf)
            pltpu.async_copy(feat_hbm.at[parbuf], parrows, sem).wait()
            for c in range(NCH):
                s = parrows[0, pl.ds(c * _L, _L)]
                for r in range(1, parrows.shape[0]):
                    s = s + parrows[r, pl.ds(c * _L, _L)]
                outbuf[pl.ds(c * _L, _L)] = s
            pltpu.sync_copy(outbuf, final.at[pl.ds(3 * D, D)])

    mesh = plsc.VectorSubcoreMesh(core_axis_name="c", subcore_axis_name="s",
                                  num_cores=_NC, num_subcores=_NS)
    return pl.kernel(
        body,
        out_type=(
            jax.ShapeDtypeStruct((4 * D,), jnp.float32),
            jax.ShapeDtypeStruct((_NW * _PW,), jnp.float32),
        ),
        mesh=mesh,
        compiler_params=pltpu.CompilerParams(needs_layout_passes=False),
        scratch_types=[
            pltpu.VMEM((EPW,), jnp.int32),
            pltpu.VMEM((EPW,), jnp.int32),
            pltpu.VMEM((EPW + _L,), jnp.int32),
            pltpu.VMEM((EPW + _L,), jnp.int32),
            pltpu.VMEM((4 * D,), jnp.float32),
            pltpu.VMEM((_L,), jnp.int32),
            pltpu.VMEM((_L,), jnp.int32),
            pltpu.VMEM((_L,), jnp.int32),
            pltpu.VMEM((_L, D), jnp.float32),
            pltpu.VMEM((_L, D), jnp.float32),
            pltpu.VMEM((2 * D,), jnp.float32),
            pltpu.VMEM((_L,), jnp.float32),
            pltpu.VMEM((8,), jnp.int32),
            pltpu.VMEM((8, D), jnp.float32),
            pltpu.VMEM((D * D,), jnp.float32),
            pltpu.VMEM((D,), jnp.float32),
            pltpu.VMEM((_NW * _PW,), jnp.float32),
            pltpu.VMEM((D + _L,), jnp.float32),
            pltpu.VMEM((D,), jnp.float32),
            pltpu.SMEM((8,), jnp.float32),
            pltpu.SMEM((8,), jnp.int32),
            pltpu.SemaphoreType.DMA,
            pltpu.SemaphoreType.DMA,
        ],
    )


def kernel(feat, edge_index, op, parallel, W_f, al_f, ar_f, b_f,
           W_b, al_b, ar_b, b_b):
    N, D = feat.shape
    E = edge_index.shape[1]
    H = al_f.shape[0]
    assert H == 1 and D == 128

    wv = jnp.zeros((4, D), jnp.float32)

    src = edge_index[0]
    dst = edge_index[1]
    op_arr = jnp.full((_L,), op, dtype=jnp.int32)
    par = parallel.astype(jnp.int32)

    final, _ = _make_scan(E, N, D)(
        src, dst, feat, wv.reshape(4 * D), op_arr, par,
        W_f.reshape(D * D), W_b.reshape(D * D), b_f, b_b)
    return final


# speedup vs baseline: 1.0440x; 1.0203x over previous
import functools

import jax
import jax.numpy as jnp
from jax import lax
from jax.experimental import pallas as pl
from jax.experimental.pallas import tpu as pltpu
from jax.experimental.pallas import tpu_sc as plsc

_NC = 1
_NS = 16
_NW = _NC * _NS
_L = 16
_NEG = -1.0e30
_PW = 272


def _prep_body(alf, arf, alb, arb, wf, wb, out):
    dn = (((1,), (1,)), ((), ()))
    out[0:1, :] = lax.dot_general(alf[:], wf[:], dn, preferred_element_type=jnp.float32)
    out[1:2, :] = lax.dot_general(arf[:], wf[:], dn, preferred_element_type=jnp.float32)
    out[2:3, :] = lax.dot_general(alb[:], wb[:], dn, preferred_element_type=jnp.float32)
    out[3:4, :] = lax.dot_general(arb[:], wb[:], dn, preferred_element_type=jnp.float32)


def _make_scan(E, N, D):
    EPW = E // _NW
    BLK = 25
    NB = EPW // (BLK * _L)
    assert EPW * _NW == E and NB * BLK * _L == EPW and D == 128
    NCH = D // _L

    def body(src_hbm, dst_hbm, feat_hbm, wv_hbm, op_hbm, par_hbm,
             wf_hbm, wb_hbm, bf_hbm, bb_hbm,
             final, parts,
             srcb, dstb, flist_f, flist_b, wvbuf, opbuf, idxs, idxd,
             rows_s, rows_d, accbuf, mdbuf, parbuf, parrows,
             wmat, bbuf, pbuf, wbuf, outbuf, scal, cnts, sem, sem2):
        tid = lax.axis_index("s")
        base = tid * EPW

        stage1 = pltpu.async_copy(src_hbm.at[pl.ds(base, EPW)], srcb, sem2)
        stage2 = pltpu.async_copy(dst_hbm.at[pl.ds(base, EPW)], dstb, sem2)
        pltpu.sync_copy(wv_hbm, wvbuf)
        pltpu.sync_copy(op_hbm, opbuf)

        opv = opbuf[...]
        opn = opv[0]
        idxs[:] = opv
        pltpu.async_copy(feat_hbm.at[idxs], rows_s, sem).wait()

        def dot_row(ref_a, ia, wbase):
            a = ref_a[ia, pl.ds(0, _L)] * wvbuf[pl.ds(wbase, _L)]
            for c in range(1, NCH):
                a = a + ref_a[ia, pl.ds(c * _L, _L)] * wvbuf[pl.ds(wbase + c * _L, _L)]
            return jnp.sum(a)

        er_f = dot_row(rows_s, 0, D)
        er_b = dot_row(rows_s, 0, 3 * D)

        @pl.when(tid == 2)
        def _():
            for c in range(NCH):
                outbuf[pl.ds(c * _L, _L)] = rows_s[0, pl.ds(c * _L, _L)]

        scal[0] = _NEG
        scal[1] = 0.0
        scal[2] = _NEG
        scal[3] = 0.0
        cnts[0] = 0
        cnts[1] = 0
        for c in range(2 * NCH):
            accbuf[pl.ds(c * _L, _L)] = jnp.zeros((_L,), jnp.float32)

        def exp_scalar(x):
            return jnp.max(jnp.exp(jnp.full((_L,), x, jnp.float32)))

        def online_update(dirn, s, rowref, j):
            m_old = scal[2 * dirn]
            d_old = scal[2 * dirn + 1]

            @pl.when(s <= m_old)
            def _():
                w = exp_scalar(s - m_old)
                scal[2 * dirn + 1] = d_old + w
                for c in range(NCH):
                    ds_ = pl.ds(dirn * D + c * _L, _L)
                    accbuf[ds_] = accbuf[ds_] + w * rowref[j, pl.ds(c * _L, _L)]

            @pl.when(s > m_old)
            def _():
                sc = exp_scalar(m_old - s)
                scal[2 * dirn] = s
                scal[2 * dirn + 1] = d_old * sc + 1.0
                for c in range(NCH):
                    ds_ = pl.ds(dirn * D + c * _L, _L)
                    accbuf[ds_] = accbuf[ds_] * sc + rowref[j, pl.ds(c * _L, _L)]

        stage1.wait()
        stage2.wait()

        def record_vreg(off):
            sv = srcb[pl.ds(off, _L)]
            dv = dstb[pl.ds(off, _L)]
            mf = dv == opv
            mb = sv == opv
            cf = cnts[0]
            cb = cnts[1]
            plsc.store_compressed(flist_f.at[pl.ds(cf, _L)], sv, mask=mf)
            plsc.store_compressed(flist_b.at[pl.ds(cb, _L)], dv, mask=mb)
            cnts[0] = cf + plsc.all_reduce_population_count(mf)[0]
            cnts[1] = cb + plsc.all_reduce_population_count(mb)[0]

        def block(b, carry):
            bbase = b * (BLK * _L)
            hit = srcb[pl.ds(bbase, _L)] == opv
            hit = hit | (dstb[pl.ds(bbase, _L)] == opv)
            for u in range(1, BLK):
                hit = hit | (srcb[pl.ds(bbase + u * _L, _L)] == opv)
                hit = hit | (dstb[pl.ds(bbase + u * _L, _L)] == opv)
            cnt = jnp.sum(jnp.where(hit, 1, 0))

            @pl.when(cnt > 0)
            def _():
                def rescan(u, c2):
                    record_vreg(bbase + u * _L)
                    return c2
                lax.fori_loop(0, BLK, rescan, 0)
            return carry

        lax.fori_loop(0, NB, block, 0)

        lane = lax.broadcasted_iota(jnp.int32, (_L,), 0)

        def post_pass(dirn, flist, idxbuf, rows, wbase, er):
            cnt = cnts[dirn]

            def chunk(ci, carry):
                i = ci * _L
                raw = flist[pl.ds(i, _L)]
                valid = (lane + i) < jnp.full((_L,), cnt, jnp.int32)
                idxbuf[:] = jnp.where(valid, raw, 0)
                pltpu.async_copy(feat_hbm.at[idxbuf], rows, sem).wait()

                def lane_body(j, c3):
                    @pl.when(i + j < cnt)
                    def _():
                        x = dot_row(rows, j, wbase) + er
                        online_update(dirn, jnp.where(x >= 0.0, x, 0.2 * x),
                                      rows, j)
                    return c3

                lax.fori_loop(0, _L, lane_body, 0)
                return carry

            lax.fori_loop(0, (cnt + _L - 1) // _L, chunk, 0)

        post_pass(0, flist_f, idxs, rows_s, 0, er_f)
        post_pass(1, flist_b, idxd, rows_d, 2 * D, er_b)

        mdv = jnp.zeros((_L,), jnp.float32)
        for k in range(4):
            mdv = jnp.where(lane == k, scal[k], mdv)
        mdbuf[:] = mdv
        pltpu.sync_copy(mdbuf, parts.at[pl.ds(tid * _PW, _L)])
        pltpu.sync_copy(accbuf, parts.at[pl.ds(tid * _PW + _L, 2 * D)])
        plsc.subcore_barrier()

        @pl.when(tid < 2)
        def _():
            @pl.when(tid == 0)
            def _():
                pltpu.sync_copy(wf_hbm, wmat)
                pltpu.sync_copy(bf_hbm, bbuf)

            @pl.when(tid == 1)
            def _():
                pltpu.sync_copy(wb_hbm, wmat)
                pltpu.sync_copy(bb_hbm, bbuf)

            pltpu.sync_copy(parts, pbuf)
            accoff = _L + tid * D

            ms, ds_l = [], []
            for t in range(_NW):
                mv = pbuf[pl.ds(t * _PW, _L)]
                ms.append(jnp.where(tid == 0, mv[0], mv[2]))
                ds_l.append(jnp.where(tid == 0, mv[1], mv[3]))
            mx = ms[0]
            for t in range(1, _NW):
                mx = jnp.maximum(mx, ms[t])
            ws = [exp_scalar(ms[t] - mx) for t in range(_NW)]
            dtot = ds_l[0] * ws[0]
            for t in range(1, _NW):
                dtot = dtot + ds_l[t] * ws[t]
            dvec = jnp.full((_L,), dtot, jnp.float32)
            inv = jnp.where(dvec > 0.0, jnp.ones((_L,), jnp.float32) / dvec,
                            jnp.zeros((_L,), jnp.float32))
            for c in range(NCH):
                a = pbuf[pl.ds(accoff + c * _L, _L)] * ws[0]
                for t in range(1, _NW):
                    a = a + pbuf[pl.ds(t * _PW + accoff + c * _L, _L)] * ws[t]
                wbuf[pl.ds(c * _L, _L)] = a * inv

            def mv_step(k, acc):
                wk = wbuf[pl.ds(k, _L)][0]
                return tuple(acc[c] + wk * wmat[pl.ds(k * D + c * _L, _L)]
                             for c in range(NCH))

            zero = jnp.zeros((_L,), jnp.float32)
            fh = lax.fori_loop(0, D, mv_step, (zero,) * NCH)
            for c in range(NCH):
                outbuf[pl.ds(c * _L, _L)] = fh[c] + bbuf[pl.ds(c * _L, _L)]
            pltpu.sync_copy(outbuf, final.at[pl.ds(tid * D, D)])

        @pl.when(tid == 2)
        def _():
            pltpu.sync_copy(outbuf, final.at[pl.ds(2 * D, D)])

        @pl.when(tid == 3)
        def _():
            pltpu.sync_copy(par_hbm, parbuf)
            pltpu.async_copy(feat_hbm.at[parbuf], parrows, sem).wait()
            for c in range(NCH):
                s = parrows[0, pl.ds(c * _L, _L)]
                for r in range(1, parrows.shape[0]):
                    s = s + parrows[r, pl.ds(c * _L, _L)]
                outbuf[pl.ds(c * _L, _L)] = s
            pltpu.sync_copy(outbuf, final.at[pl.ds(3 * D, D)])

    mesh = plsc.VectorSubcoreMesh(core_axis_name="c", subcore_axis_name="s",
                                  num_cores=_NC, num_subcores=_NS)
    return pl.kernel(
        body,
        out_type=(
            jax.ShapeDtypeStruct((4 * D,), jnp.float32),
            jax.ShapeDtypeStruct((_NW * _PW,), jnp.float32),
        ),
        mesh=mesh,
        compiler_params=pltpu.CompilerParams(needs_layout_passes=False),
        scratch_types=[
            pltpu.VMEM((EPW,), jnp.int32),
            pltpu.VMEM((EPW,), jnp.int32),
            pltpu.VMEM((EPW + _L,), jnp.int32),
            pltpu.VMEM((EPW + _L,), jnp.int32),
            pltpu.VMEM((4 * D,), jnp.float32),
            pltpu.VMEM((_L,), jnp.int32),
            pltpu.VMEM((_L,), jnp.int32),
            pltpu.VMEM((_L,), jnp.int32),
            pltpu.VMEM((_L, D), jnp.float32),
            pltpu.VMEM((_L, D), jnp.float32),
            pltpu.VMEM((2 * D,), jnp.float32),
            pltpu.VMEM((_L,), jnp.float32),
            pltpu.VMEM((8,), jnp.int32),
            pltpu.VMEM((8, D), jnp.float32),
            pltpu.VMEM((D * D,), jnp.float32),
            pltpu.VMEM((D,), jnp.float32),
            pltpu.VMEM((_NW * _PW,), jnp.float32),
            pltpu.VMEM((D + _L,), jnp.float32),
            pltpu.VMEM((D,), jnp.float32),
            pltpu.SMEM((8,), jnp.float32),
            pltpu.SMEM((8,), jnp.int32),
            pltpu.SemaphoreType.DMA,
            pltpu.SemaphoreType.DMA,
        ],
    )


def kernel(feat, edge_index, op, parallel, W_f, al_f, ar_f, b_f,
           W_b, al_b, ar_b, b_b):
    N, D = feat.shape
    E = edge_index.shape[1]
    H = al_f.shape[0]
    assert H == 1 and D == 128

    wv = pl.pallas_call(
        _prep_body,
        out_shape=jax.ShapeDtypeStruct((4, D), jnp.float32),
    )(al_f, ar_f, al_b, ar_b, W_f, W_b)

    src = edge_index[0]
    dst = edge_index[1]
    op_arr = jnp.full((_L,), op, dtype=jnp.int32)
    par = parallel.astype(jnp.int32)

    final, _ = _make_scan(E, N, D)(
        src, dst, feat, wv.reshape(4 * D), op_arr, par,
        W_f.reshape(D * D), W_b.reshape(D * D), b_f, b_b)
    return final


# speedup vs baseline: 1.2477x; 1.1951x over previous
import functools

import jax
import jax.numpy as jnp
from jax import lax
from jax.experimental import pallas as pl
from jax.experimental.pallas import tpu as pltpu
from jax.experimental.pallas import tpu_sc as plsc

_NC = 1
_NS = 16
_NW = _NC * _NS
_L = 16
_NEG = -1.0e30
_PW = 272


def _prep_body(alf, arf, alb, arb, wf, wb, out):
    dn = (((1,), (1,)), ((), ()))
    out[0:1, :] = lax.dot_general(alf[:], wf[:], dn, preferred_element_type=jnp.float32)
    out[1:2, :] = lax.dot_general(arf[:], wf[:], dn, preferred_element_type=jnp.float32)
    out[2:3, :] = lax.dot_general(alb[:], wb[:], dn, preferred_element_type=jnp.float32)
    out[3:4, :] = lax.dot_general(arb[:], wb[:], dn, preferred_element_type=jnp.float32)


def _make_scan(E, N, D):
    EPW = E // _NW
    BLK = 25
    NB = EPW // (BLK * _L)
    assert EPW * _NW == E and NB * BLK * _L == EPW and D == 128
    NCH = D // _L

    def body(ei_hbm, feat_hbm, wv_hbm, op_hbm, par_hbm,
             wf_hbm, wb_hbm, bf_hbm, bb_hbm,
             final, parts,
             srcb, dstb, flist_f, flist_b, wvbuf, opbuf, idxs, idxd,
             rows_s, rows_d, accbuf, mdbuf, parbuf, parrows,
             wmat, bbuf, pbuf, wbuf, outbuf, scal, cnts, sem, sem2):
        tid = lax.axis_index("s")
        base = tid * EPW

        stage1 = pltpu.async_copy(ei_hbm.at[pl.ds(base, EPW)], srcb, sem2)
        stage2 = pltpu.async_copy(ei_hbm.at[pl.ds(E + base, EPW)], dstb, sem2)
        pltpu.sync_copy(wv_hbm, wvbuf)
        pltpu.sync_copy(op_hbm, opbuf)

        opv = opbuf[...]
        opn = opv[0]
        idxs[:] = opv
        pltpu.async_copy(feat_hbm.at[idxs], rows_s, sem).wait()

        def dot_row(ref_a, ia, wbase):
            a = ref_a[ia, pl.ds(0, _L)] * wvbuf[pl.ds(wbase, _L)]
            for c in range(1, NCH):
                a = a + ref_a[ia, pl.ds(c * _L, _L)] * wvbuf[pl.ds(wbase + c * _L, _L)]
            return jnp.sum(a)

        er_f = dot_row(rows_s, 0, D)
        er_b = dot_row(rows_s, 0, 3 * D)

        @pl.when(tid == 2)
        def _():
            for c in range(NCH):
                outbuf[pl.ds(c * _L, _L)] = rows_s[0, pl.ds(c * _L, _L)]

        scal[0] = _NEG
        scal[1] = 0.0
        scal[2] = _NEG
        scal[3] = 0.0
        cnts[0] = 0
        cnts[1] = 0
        for c in range(2 * NCH):
            accbuf[pl.ds(c * _L, _L)] = jnp.zeros((_L,), jnp.float32)

        def exp_scalar(x):
            return jnp.max(jnp.exp(jnp.full((_L,), x, jnp.float32)))

        def online_update(dirn, s, rowref, j):
            m_old = scal[2 * dirn]
            d_old = scal[2 * dirn + 1]

            @pl.when(s <= m_old)
            def _():
                w = exp_scalar(s - m_old)
                scal[2 * dirn + 1] = d_old + w
                for c in range(NCH):
                    ds_ = pl.ds(dirn * D + c * _L, _L)
                    accbuf[ds_] = accbuf[ds_] + w * rowref[j, pl.ds(c * _L, _L)]

            @pl.when(s > m_old)
            def _():
                sc = exp_scalar(m_old - s)
                scal[2 * dirn] = s
                scal[2 * dirn + 1] = d_old * sc + 1.0
                for c in range(NCH):
                    ds_ = pl.ds(dirn * D + c * _L, _L)
                    accbuf[ds_] = accbuf[ds_] * sc + rowref[j, pl.ds(c * _L, _L)]

        stage1.wait()
        stage2.wait()

        def record_vreg(off):
            sv = srcb[pl.ds(off, _L)]
            dv = dstb[pl.ds(off, _L)]
            mf = dv == opv
            mb = sv == opv
            cf = cnts[0]
            cb = cnts[1]
            plsc.store_compressed(flist_f.at[pl.ds(cf, _L)], sv, mask=mf)
            plsc.store_compressed(flist_b.at[pl.ds(cb, _L)], dv, mask=mb)
            cnts[0] = cf + plsc.all_reduce_population_count(mf)[0]
            cnts[1] = cb + plsc.all_reduce_population_count(mb)[0]

        def block(b, carry):
            bbase = b * (BLK * _L)
            hit = srcb[pl.ds(bbase, _L)] == opv
            hit = hit | (dstb[pl.ds(bbase, _L)] == opv)
            for u in range(1, BLK):
                hit = hit | (srcb[pl.ds(bbase + u * _L, _L)] == opv)
                hit = hit | (dstb[pl.ds(bbase + u * _L, _L)] == opv)
            cnt = jnp.sum(jnp.where(hit, 1, 0))

            @pl.when(cnt > 0)
            def _():
                def rescan(u, c2):
                    record_vreg(bbase + u * _L)
                    return c2
                lax.fori_loop(0, BLK, rescan, 0)
            return carry

        lax.fori_loop(0, NB, block, 0)

        lane = lax.broadcasted_iota(jnp.int32, (_L,), 0)

        def post_pass(dirn, flist, idxbuf, rows, wbase, er):
            cnt = cnts[dirn]

            def chunk(ci, carry):
                i = ci * _L
                raw = flist[pl.ds(i, _L)]
                valid = (lane + i) < jnp.full((_L,), cnt, jnp.int32)
                idxbuf[:] = jnp.where(valid, raw, 0)
                pltpu.async_copy(feat_hbm.at[idxbuf], rows, sem).wait()

                def lane_body(j, c3):
                    @pl.when(i + j < cnt)
                    def _():
                        x = dot_row(rows, j, wbase) + er
                        online_update(dirn, jnp.where(x >= 0.0, x, 0.2 * x),
                                      rows, j)
                    return c3

                lax.fori_loop(0, _L, lane_body, 0)
                return carry

            lax.fori_loop(0, (cnt + _L - 1) // _L, chunk, 0)

        post_pass(0, flist_f, idxs, rows_s, 0, er_f)
        post_pass(1, flist_b, idxd, rows_d, 2 * D, er_b)

        mdv = jnp.zeros((_L,), jnp.float32)
        for k in range(4):
            mdv = jnp.where(lane == k, scal[k], mdv)
        mdbuf[:] = mdv
        pltpu.sync_copy(mdbuf, parts.at[pl.ds(tid * _PW, _L)])
        pltpu.sync_copy(accbuf, parts.at[pl.ds(tid * _PW + _L, 2 * D)])
        plsc.subcore_barrier()

        @pl.when(tid < 2)
        def _():
            @pl.when(tid == 0)
            def _():
                pltpu.sync_copy(wf_hbm, wmat)
                pltpu.sync_copy(bf_hbm, bbuf)

            @pl.when(tid == 1)
            def _():
                pltpu.sync_copy(wb_hbm, wmat)
                pltpu.sync_copy(bb_hbm, bbuf)

            pltpu.sync_copy(parts, pbuf)
            accoff = _L + tid * D

            ms, ds_l = [], []
            for t in range(_NW):
                mv = pbuf[pl.ds(t * _PW, _L)]
                ms.append(jnp.where(tid == 0, mv[0], mv[2]))
                ds_l.append(jnp.where(tid == 0, mv[1], mv[3]))
            mx = ms[0]
            for t in range(1, _NW):
                mx = jnp.maximum(mx, ms[t])
            ws = [exp_scalar(ms[t] - mx) for t in range(_NW)]
            dtot = ds_l[0] * ws[0]
            for t in range(1, _NW):
                dtot = dtot + ds_l[t] * ws[t]
            dvec = jnp.full((_L,), dtot, jnp.float32)
            inv = jnp.where(dvec > 0.0, jnp.ones((_L,), jnp.float32) / dvec,
                            jnp.zeros((_L,), jnp.float32))
            for c in range(NCH):
                a = pbuf[pl.ds(accoff + c * _L, _L)] * ws[0]
                for t in range(1, _NW):
                    a = a + pbuf[pl.ds(t * _PW + accoff + c * _L, _L)] * ws[t]
                wbuf[pl.ds(c * _L, _L)] = a * inv

            def mv_step(k, acc):
                wk = wbuf[pl.ds(k, _L)][0]
                return tuple(acc[c] + wk * wmat[pl.ds(k * D + c * _L, _L)]
                             for c in range(NCH))

            zero = jnp.zeros((_L,), jnp.float32)
            fh = lax.fori_loop(0, D, mv_step, (zero,) * NCH)
            for c in range(NCH):
                outbuf[pl.ds(c * _L, _L)] = fh[c] + bbuf[pl.ds(c * _L, _L)]
            pltpu.sync_copy(outbuf, final.at[pl.ds(tid * D, D)])

        @pl.when(tid == 2)
        def _():
            pltpu.sync_copy(outbuf, final.at[pl.ds(2 * D, D)])

        @pl.when(tid == 3)
        def _():
            pltpu.sync_copy(par_hbm, parbuf)
            pltpu.async_copy(feat_hbm.at[parbuf], parrows, sem).wait()
            for c in range(NCH):
                s = parrows[0, pl.ds(c * _L, _L)]
                for r in range(1, parrows.shape[0]):
                    s = s + parrows[r, pl.ds(c * _L, _L)]
                outbuf[pl.ds(c * _L, _L)] = s
            pltpu.sync_copy(outbuf, final.at[pl.ds(3 * D, D)])

    mesh = plsc.VectorSubcoreMesh(core_axis_name="c", subcore_axis_name="s",
                                  num_cores=_NC, num_subcores=_NS)
    return pl.kernel(
        body,
        out_type=(
            jax.ShapeDtypeStruct((4 * D,), jnp.float32),
            jax.ShapeDtypeStruct((_NW * _PW,), jnp.float32),
        ),
        mesh=mesh,
        compiler_params=pltpu.CompilerParams(needs_layout_passes=False),
        scratch_types=[
            pltpu.VMEM((EPW,), jnp.int32),
            pltpu.VMEM((EPW,), jnp.int32),
            pltpu.VMEM((EPW + _L,), jnp.int32),
            pltpu.VMEM((EPW + _L,), jnp.int32),
            pltpu.VMEM((4 * D,), jnp.float32),
            pltpu.VMEM((_L,), jnp.int32),
            pltpu.VMEM((_L,), jnp.int32),
            pltpu.VMEM((_L,), jnp.int32),
            pltpu.VMEM((_L, D), jnp.float32),
            pltpu.VMEM((_L, D), jnp.float32),
            pltpu.VMEM((2 * D,), jnp.float32),
            pltpu.VMEM((_L,), jnp.float32),
            pltpu.VMEM((8,), jnp.int32),
            pltpu.VMEM((8, D), jnp.float32),
            pltpu.VMEM((D * D,), jnp.float32),
            pltpu.VMEM((D,), jnp.float32),
            pltpu.VMEM((_NW * _PW,), jnp.float32),
            pltpu.VMEM((D + _L,), jnp.float32),
            pltpu.VMEM((D,), jnp.float32),
            pltpu.SMEM((8,), jnp.float32),
            pltpu.SMEM((8,), jnp.int32),
            pltpu.SemaphoreType.DMA,
            pltpu.SemaphoreType.DMA,
        ],
    )


def kernel(feat, edge_index, op, parallel, W_f, al_f, ar_f, b_f,
           W_b, al_b, ar_b, b_b):
    N, D = feat.shape
    E = edge_index.shape[1]
    H = al_f.shape[0]
    assert H == 1 and D == 128

    wv = pl.pallas_call(
        _prep_body,
        out_shape=jax.ShapeDtypeStruct((4, D), jnp.float32),
    )(al_f, ar_f, al_b, ar_b, W_f, W_b)

    op_arr = jnp.full((_L,), op, dtype=jnp.int32)
    par = parallel.astype(jnp.int32)

    final, _ = _make_scan(E, N, D)(
        edge_index.reshape(2 * E), feat, wv.reshape(4 * D), op_arr, par,
        W_f.reshape(D * D), W_b.reshape(D * D), b_f, b_b)
    return final


# speedup vs baseline: 1.2847x; 1.0297x over previous
import functools

import jax
import jax.numpy as jnp
from jax import lax
from jax.experimental import pallas as pl
from jax.experimental.pallas import tpu as pltpu
from jax.experimental.pallas import tpu_sc as plsc

_NC = 1
_NS = 16
_NW = _NC * _NS
_L = 16
_NEG = -1.0e30
_PW = 272


def _prep_body(alf, arf, alb, arb, wf, wb, out):
    dn = (((1,), (1,)), ((), ()))
    out[0:1, :] = lax.dot_general(alf[:], wf[:], dn, preferred_element_type=jnp.float32)
    out[1:2, :] = lax.dot_general(arf[:], wf[:], dn, preferred_element_type=jnp.float32)
    out[2:3, :] = lax.dot_general(alb[:], wb[:], dn, preferred_element_type=jnp.float32)
    out[3:4, :] = lax.dot_general(arb[:], wb[:], dn, preferred_element_type=jnp.float32)


def _make_scan(E, N, D):
    EPW = E // _NW
    BLK = 25
    NB = EPW // (BLK * _L)
    assert EPW * _NW == E and NB * BLK * _L == EPW and D == 128
    NCH = D // _L

    def body(ei_hbm, feat_hbm, wv_hbm, op_hbm, par_hbm,
             wf_hbm, wb_hbm, bf_hbm, bb_hbm,
             final, parts,
             srcb, dstb, flist_f, flist_b, wvbuf, opbuf, idxs, idxd,
             rows_s, rows_d, accbuf, mdbuf, parbuf, parrows,
             wmat, bbuf, pbuf, wbuf, outbuf, scal, cnts, sem, sem2):
        tid = lax.axis_index("s")
        base = tid * EPW

        stage1 = pltpu.async_copy(ei_hbm.at[pl.ds(base, EPW)], srcb, sem2)
        stage2 = pltpu.async_copy(ei_hbm.at[pl.ds(E + base, EPW)], dstb, sem2)
        pltpu.sync_copy(wv_hbm, wvbuf)
        pltpu.sync_copy(op_hbm, opbuf)

        opv = opbuf[...]
        opn = opv[0]
        idxs[:] = opv
        pltpu.async_copy(feat_hbm.at[idxs], rows_s, sem).wait()

        def dot_row(ref_a, ia, wbase):
            a = ref_a[ia, pl.ds(0, _L)] * wvbuf[pl.ds(wbase, _L)]
            for c in range(1, NCH):
                a = a + ref_a[ia, pl.ds(c * _L, _L)] * wvbuf[pl.ds(wbase + c * _L, _L)]
            return jnp.sum(a)

        er_f = dot_row(rows_s, 0, D)
        er_b = dot_row(rows_s, 0, 3 * D)

        @pl.when(tid == 2)
        def _():
            for c in range(NCH):
                outbuf[pl.ds(c * _L, _L)] = rows_s[0, pl.ds(c * _L, _L)]

        scal[0] = _NEG
        scal[1] = 0.0
        scal[2] = _NEG
        scal[3] = 0.0
        cnts[0] = 0
        cnts[1] = 0
        for c in range(2 * NCH):
            accbuf[pl.ds(c * _L, _L)] = jnp.zeros((_L,), jnp.float32)

        def exp_scalar(x):
            return jnp.max(jnp.exp(jnp.full((_L,), x, jnp.float32)))

        def online_update(dirn, s, rowref, j):
            m_old = scal[2 * dirn]
            d_old = scal[2 * dirn + 1]

            @pl.when(s <= m_old)
            def _():
                w = exp_scalar(s - m_old)
                scal[2 * dirn + 1] = d_old + w
                for c in range(NCH):
                    ds_ = pl.ds(dirn * D + c * _L, _L)
                    accbuf[ds_] = accbuf[ds_] + w * rowref[j, pl.ds(c * _L, _L)]

            @pl.when(s > m_old)
            def _():
                sc = exp_scalar(m_old - s)
                scal[2 * dirn] = s
                scal[2 * dirn + 1] = d_old * sc + 1.0
                for c in range(NCH):
                    ds_ = pl.ds(dirn * D + c * _L, _L)
                    accbuf[ds_] = accbuf[ds_] * sc + rowref[j, pl.ds(c * _L, _L)]

        stage1.wait()
        stage2.wait()

        @pl.when(tid == 0)
        def _():
            pltpu.async_copy(wf_hbm, wmat, sem2)
            pltpu.async_copy(bf_hbm, bbuf, sem2)

        @pl.when(tid == 1)
        def _():
            pltpu.async_copy(wb_hbm, wmat, sem2)
            pltpu.async_copy(bb_hbm, bbuf, sem2)

        def record_vreg(off):
            sv = srcb[pl.ds(off, _L)]
            dv = dstb[pl.ds(off, _L)]
            mf = dv == opv
            mb = sv == opv
            cf = cnts[0]
            cb = cnts[1]
            plsc.store_compressed(flist_f.at[pl.ds(cf, _L)], sv, mask=mf)
            plsc.store_compressed(flist_b.at[pl.ds(cb, _L)], dv, mask=mb)
            cnts[0] = cf + plsc.all_reduce_population_count(mf)[0]
            cnts[1] = cb + plsc.all_reduce_population_count(mb)[0]

        def block(b, carry):
            bbase = b * (BLK * _L)
            hit = srcb[pl.ds(bbase, _L)] == opv
            hit = hit | (dstb[pl.ds(bbase, _L)] == opv)
            for u in range(1, BLK):
                hit = hit | (srcb[pl.ds(bbase + u * _L, _L)] == opv)
                hit = hit | (dstb[pl.ds(bbase + u * _L, _L)] == opv)
            cnt = jnp.sum(jnp.where(hit, 1, 0))

            @pl.when(cnt > 0)
            def _():
                def rescan(u, c2):
                    record_vreg(bbase + u * _L)
                    return c2
                lax.fori_loop(0, BLK, rescan, 0)
            return carry

        lax.fori_loop(0, NB, block, 0)

        lane = lax.broadcasted_iota(jnp.int32, (_L,), 0)

        def post_pass(dirn, flist, idxbuf, rows, wbase, er):
            cnt = cnts[dirn]

            def chunk(ci, carry):
                i = ci * _L
                raw = flist[pl.ds(i, _L)]
                valid = (lane + i) < jnp.full((_L,), cnt, jnp.int32)
                idxbuf[:] = jnp.where(valid, raw, 0)
                pltpu.async_copy(feat_hbm.at[idxbuf], rows, sem).wait()

                def lane_body(j, c3):
                    @pl.when(i + j < cnt)
                    def _():
                        x = dot_row(rows, j, wbase) + er
                        online_update(dirn, jnp.where(x >= 0.0, x, 0.2 * x),
                                      rows, j)
                    return c3

                lax.fori_loop(0, _L, lane_body, 0)
                return carry

            lax.fori_loop(0, (cnt + _L - 1) // _L, chunk, 0)

        post_pass(0, flist_f, idxs, rows_s, 0, er_f)
        post_pass(1, flist_b, idxd, rows_d, 2 * D, er_b)

        mdv = jnp.zeros((_L,), jnp.float32)
        for k in range(4):
            mdv = jnp.where(lane == k, scal[k], mdv)
        mdbuf[:] = mdv
        pltpu.sync_copy(mdbuf, parts.at[pl.ds(tid * _PW, _L)])
        pltpu.sync_copy(accbuf, parts.at[pl.ds(tid * _PW + _L, 2 * D)])
        plsc.subcore_barrier()

        @pl.when(tid < 2)
        def _():
            pltpu.make_async_copy(wf_hbm, wmat, sem2).wait()
            pltpu.make_async_copy(bf_hbm, bbuf, sem2).wait()
            pltpu.sync_copy(parts, pbuf)
            accoff = _L + tid * D

            ms, ds_l = [], []
            for t in range(_NW):
                mv = pbuf[pl.ds(t * _PW, _L)]
                ms.append(jnp.where(tid == 0, mv[0], mv[2]))
                ds_l.append(jnp.where(tid == 0, mv[1], mv[3]))
            mx = ms[0]
            for t in range(1, _NW):
                mx = jnp.maximum(mx, ms[t])
            ws = [exp_scalar(ms[t] - mx) for t in range(_NW)]
            dtot = ds_l[0] * ws[0]
            for t in range(1, _NW):
                dtot = dtot + ds_l[t] * ws[t]
            dvec = jnp.full((_L,), dtot, jnp.float32)
            inv = jnp.where(dvec > 0.0, jnp.ones((_L,), jnp.float32) / dvec,
                            jnp.zeros((_L,), jnp.float32))
            for c in range(NCH):
                a = pbuf[pl.ds(accoff + c * _L, _L)] * ws[0]
                for t in range(1, _NW):
                    a = a + pbuf[pl.ds(t * _PW + accoff + c * _L, _L)] * ws[t]
                wbuf[pl.ds(c * _L, _L)] = a * inv

            def mv_step(k, acc):
                wk = wbuf[pl.ds(k, _L)][0]
                return tuple(acc[c] + wk * wmat[pl.ds(k * D + c * _L, _L)]
                             for c in range(NCH))

            zero = jnp.zeros((_L,), jnp.float32)
            fh = lax.fori_loop(0, D, mv_step, (zero,) * NCH)
            for c in range(NCH):
                outbuf[pl.ds(c * _L, _L)] = fh[c] + bbuf[pl.ds(c * _L, _L)]
            pltpu.sync_copy(outbuf, final.at[pl.ds(tid * D, D)])

        @pl.when(tid == 2)
        def _():
            pltpu.sync_copy(outbuf, final.at[pl.ds(2 * D, D)])

        @pl.when(tid == 3)
        def _():
            pltpu.sync_copy(par_hbm, parbuf)
            pltpu.async_copy(feat_hbm.at[parbuf], parrows, sem).wait()
            for c in range(NCH):
                s = parrows[0, pl.ds(c * _L, _L)]
                for r in range(1, parrows.shape[0]):
                    s = s + parrows[r, pl.ds(c * _L, _L)]
                outbuf[pl.ds(c * _L, _L)] = s
            pltpu.sync_copy(outbuf, final.at[pl.ds(3 * D, D)])

    mesh = plsc.VectorSubcoreMesh(core_axis_name="c", subcore_axis_name="s",
                                  num_cores=_NC, num_subcores=_NS)
    return pl.kernel(
        body,
        out_type=(
            jax.ShapeDtypeStruct((4 * D,), jnp.float32),
            jax.ShapeDtypeStruct((_NW * _PW,), jnp.float32),
        ),
        mesh=mesh,
        compiler_params=pltpu.CompilerParams(needs_layout_passes=False),
        scratch_types=[
            pltpu.VMEM((EPW,), jnp.int32),
            pltpu.VMEM((EPW,), jnp.int32),
            pltpu.VMEM((EPW + _L,), jnp.int32),
            pltpu.VMEM((EPW + _L,), jnp.int32),
            pltpu.VMEM((4 * D,), jnp.float32),
            pltpu.VMEM((_L,), jnp.int32),
            pltpu.VMEM((_L,), jnp.int32),
            pltpu.VMEM((_L,), jnp.int32),
            pltpu.VMEM((_L, D), jnp.float32),
            pltpu.VMEM((_L, D), jnp.float32),
            pltpu.VMEM((2 * D,), jnp.float32),
            pltpu.VMEM((_L,), jnp.float32),
            pltpu.VMEM((8,), jnp.int32),
            pltpu.VMEM((8, D), jnp.float32),
            pltpu.VMEM((D * D,), jnp.float32),
            pltpu.VMEM((D,), jnp.float32),
            pltpu.VMEM((_NW * _PW,), jnp.float32),
            pltpu.VMEM((D + _L,), jnp.float32),
            pltpu.VMEM((D,), jnp.float32),
            pltpu.SMEM((8,), jnp.float32),
            pltpu.SMEM((8,), jnp.int32),
            pltpu.SemaphoreType.DMA,
            pltpu.SemaphoreType.DMA,
        ],
    )


def kernel(feat, edge_index, op, parallel, W_f, al_f, ar_f, b_f,
           W_b, al_b, ar_b, b_b):
    N, D = feat.shape
    E = edge_index.shape[1]
    H = al_f.shape[0]
    assert H == 1 and D == 128

    wv = pl.pallas_call(
        _prep_body,
        out_shape=jax.ShapeDtypeStruct((4, D), jnp.float32),
    )(al_f, ar_f, al_b, ar_b, W_f, W_b)

    op_arr = jnp.full((_L,), op, dtype=jnp.int32)
    par = parallel.astype(jnp.int32)

    final, _ = _make_scan(E, N, D)(
        edge_index.reshape(2 * E), feat, wv.reshape(4 * D), op_arr, par,
        W_f.reshape(D * D), W_b.reshape(D * D), b_f, b_b)
    return final
